# baseline (speedup 1.0000x reference)
import jax
import jax.numpy as jnp
from jax import lax
from jax.experimental import pallas as pl
from jax.experimental.pallas import tpu as pltpu
from jax.experimental.pallas import tpu_sc as plsc

B = 16
C = 12288
SEG = 3072
NSEG = C // SEG
CHUNK = 192
ITERS = SEG // CHUNK


def _body(coords_hbm, na_hbm, out_hbm, buf0, buf1, na_v, out_v, sem0, sem1):
    c = lax.axis_index("c")
    s = lax.axis_index("s")

    @pl.when(s < 8)
    def _():
        b = c * 8 + s
        bufs = (buf0, buf1)
        sems = (sem0, sem1)

        copies = [None] * NSEG
        copies[0] = pltpu.async_copy(
            coords_hbm.at[b, pl.ds(0, SEG)], bufs[0], sems[0])
        pltpu.sync_copy(na_hbm, na_v)

        iota = lax.iota(jnp.int32, 16)
        nvec = na_v[...]
        bvec = jnp.zeros((16,), jnp.int32) + b
        n_vec = nvec.at[bvec].get(mode="promise_in_bounds")
        thr = 3 * n_vec

        zero = jnp.zeros((16,), jnp.float32)
        accs = (zero, zero, zero)
        for seg in range(NSEG):
            if seg + 1 < NSEG:
                copies[seg + 1] = pltpu.async_copy(
                    coords_hbm.at[b, pl.ds((seg + 1) * SEG, SEG)],
                    bufs[(seg + 1) % 2], sems[(seg + 1) % 2])
            copies[seg].wait()
            buf = bufs[seg % 2]
            seg_base = seg * SEG

            def body(i, accs, buf=buf, seg_base=seg_base):
                base = i * CHUNK
                acc = list(accs)
                for j in range(CHUNK // 16):
                    off = base + j * 16
                    v = buf[pl.ds(off, 16)]
                    m = (iota + (seg_base + off)) < thr
                    acc[j % 3] = acc[j % 3] + jnp.where(m, v, 0.0)
                return tuple(acc)

            accs = lax.fori_loop(0, ITERS, body, accs)

        out_v[...] = zero
        for j in range(3):
            comp = (iota + j * 16) % 3
            plsc.addupdate_scatter(out_v, [comp], accs[j])

        nf = n_vec.astype(jnp.float32)
        out_v[...] = out_v[...] / nf
        pltpu.sync_copy(out_v, out_hbm.at[b])


@jax.jit
def _center(input_coords, num_atoms):
    mesh = plsc.VectorSubcoreMesh(core_axis_name="c", subcore_axis_name="s")
    padded = pl.kernel(
        _body,
        mesh=mesh,
        out_type=jax.ShapeDtypeStruct((B, 16), jnp.float32),
        scratch_types=[
            pltpu.VMEM((SEG,), jnp.float32),
            pltpu.VMEM((SEG,), jnp.float32),
            pltpu.VMEM((16,), jnp.int32),
            pltpu.VMEM((16,), jnp.float32),
            pltpu.SemaphoreType.DMA,
            pltpu.SemaphoreType.DMA,
        ],
        compiler_params=pltpu.CompilerParams(needs_layout_passes=False),
    )(input_coords, num_atoms)
    return padded[:, :3]


def kernel(input_coords, num_atoms):
    return _center(input_coords, num_atoms.astype(jnp.int32))


# speedup vs baseline: 1.0443x; 1.0443x over previous
import jax
import jax.numpy as jnp
from jax import lax
from jax.experimental import pallas as pl
from jax.experimental.pallas import tpu as pltpu
from jax.experimental.pallas import tpu_sc as plsc

B = 16
C = 12288
CHUNK = 192
ITERS = C // CHUNK


def _body(coords_hbm, na_hbm, out_hbm, buf, na_v, out_v, sem):
    c = lax.axis_index("c")
    s = lax.axis_index("s")

    @pl.when(s < 8)
    def _():
        b = c * 8 + s
        row_cp = pltpu.async_copy(coords_hbm.at[b], buf, sem)
        pltpu.sync_copy(na_hbm, na_v)

        iota = lax.iota(jnp.int32, 16)
        nvec = na_v[...]
        bvec = jnp.zeros((16,), jnp.int32) + b
        n_vec = nvec.at[bvec].get(mode="promise_in_bounds")
        thr = 3 * n_vec
        row_cp.wait()

        def body(i, accs):
            base = i * CHUNK
            acc = list(accs)
            for j in range(CHUNK // 16):
                off = base + j * 16
                v = buf[pl.ds(off, 16)]
                m = (iota + off) < thr
                acc[j % 3] = acc[j % 3] + jnp.where(m, v, 0.0)
            return tuple(acc)

        zero = jnp.zeros((16,), jnp.float32)
        accs = lax.fori_loop(0, ITERS, body, (zero, zero, zero))

        out_v[...] = zero
        for j in range(3):
            comp = (iota + j * 16) % 3
            plsc.addupdate_scatter(out_v, [comp], accs[j])

        nf = n_vec.astype(jnp.float32)
        out_v[...] = out_v[...] / nf
        pltpu.sync_copy(out_v, out_hbm.at[b])


@jax.jit
def _center(input_coords, num_atoms):
    mesh = plsc.VectorSubcoreMesh(core_axis_name="c", subcore_axis_name="s")
    padded = pl.kernel(
        _body,
        mesh=mesh,
        out_type=jax.ShapeDtypeStruct((B, 16), jnp.float32),
        scratch_types=[
            pltpu.VMEM((C,), jnp.float32),
            pltpu.VMEM((16,), jnp.int32),
            pltpu.VMEM((16,), jnp.float32),
            pltpu.SemaphoreType.DMA,
        ],
        compiler_params=pltpu.CompilerParams(needs_layout_passes=False),
    )(input_coords, num_atoms)
    return padded[:, :3]


def kernel(input_coords, num_atoms):
    return _center(input_coords, num_atoms.astype(jnp.int32))
